# baseline (speedup 1.0000x reference)
import functools

import jax
import jax.numpy as jnp
from jax import lax
from jax.experimental import pallas as pl
from jax.experimental.pallas import tpu as pltpu
from jax.experimental.pallas import tpu_sc as plsc

B = 16384
D = 64
NF = 6
NC, NS = 2, 16
NW = NC * NS
R = B // NW
C = 128
NCH = R // C
L = 16
VOCABS = (7, 24, 2, 100, 12, 31)
TBL_OFF = (0, 7, 31, 33, 133, 145)
VTOT = 176


def kernel(dayofweek, time, sex, age, month, day,
           W_dayofweek, W_time, W_sex, W_age, W_month, W_day):
    tbl = jnp.concatenate([W_dayofweek, W_time, W_sex, W_age,
                           W_month, W_day], axis=0)

    mesh = plsc.VectorSubcoreMesh(
        core_axis_name="c", subcore_axis_name="s",
        num_cores=NC, num_subcores=NS)

    @functools.partial(
        pl.kernel,
        out_type=jax.ShapeDtypeStruct((B, NF * D), jnp.float32),
        mesh=mesh,
        compiler_params=pltpu.CompilerParams(needs_layout_passes=False),
        scratch_types=[
            pltpu.VMEM((NF * R,), jnp.int32),
            pltpu.VMEM((VTOT, D), jnp.float32),
            pltpu.VMEM((C, NF * D), jnp.float32),
            pltpu.VMEM((C, NF * D), jnp.float32),
            pltpu.SemaphoreType.DMA,
            pltpu.SemaphoreType.DMA,
            pltpu.SemaphoreType.DMA,
        ],
    )
    def sck(i0, i1, i2, i3, i4, i5, tbl_h,
            out, raw_v, tbl_v, asm_a, asm_b, bsem, w0, w1):
        wid = lax.axis_index("s") * NC + lax.axis_index("c")
        base = wid * R
        idxs = (i0, i1, i2, i3, i4, i5)
        asms = (asm_a, asm_b)
        wsem = (w0, w1)

        bc = pltpu.async_copy(tbl_h, tbl_v, bsem)
        stage = [pltpu.async_copy(idxs[f].at[pl.ds(base, R)],
                                  raw_v.at[pl.ds(f * R, R)], bsem)
                 for f in range(NF)]
        bc.wait()
        for cp in stage:
            cp.wait()

        iota = lax.iota(jnp.int32, L)
        writes = [None] * NCH
        for c in range(NCH):
            s = c % 2
            if c >= 2:
                writes[c - 2].wait()
            asm = asms[s]
            for f in range(NF):
                rbase = f * R + c * C

                def body(j, carry, _f=f, _rbase=rbase, _asm=asm):
                    rv = raw_v[pl.ds(_rbase + j * L, L)] + TBL_OFF[_f]
                    rowv = j * L + iota
                    for k in range(D):
                        colv = jnp.full((L,), k, jnp.int32)
                        v = plsc.load_gather(tbl_v, [rv, colv])
                        plsc.store_scatter(
                            _asm, [rowv, jnp.full((L,), _f * D + k,
                                                  jnp.int32)], v)
                    return carry

                lax.fori_loop(0, C // L, body, 0)
            writes[c] = pltpu.async_copy(
                asm, out.at[pl.ds(base + c * C, C), :], wsem[s])
        writes[NCH - 2].wait()
        writes[NCH - 1].wait()

    return sck(dayofweek.astype(jnp.int32), time.astype(jnp.int32),
               sex.astype(jnp.int32), age.astype(jnp.int32),
               month.astype(jnp.int32), day.astype(jnp.int32),
               tbl)
